# baseline (speedup 1.0000x reference)
import functools

import jax
import jax.numpy as jnp
from jax import lax
from jax.experimental import pallas as pl
from jax.experimental.pallas import tpu as pltpu
from jax.experimental.pallas import tpu_sc as plsc

_B = 16384
_NW = 32
_CHUNK = _B // _NW
_L = 16
_XP_OFF = 2 * _B


def _sc_body(pidx_hbm, ptab_hbm, out_hbm, xv, xpv, av, bv, ev, outv,
             sem0, sem1, sem2, sem3):
    wid = lax.axis_index("s") * 2 + lax.axis_index("c")
    base = wid * _CHUNK
    cx = pltpu.async_copy(pidx_hbm.at[pl.ds(2 * base, 2 * _CHUNK)], xv, sem0)
    cxp = pltpu.async_copy(
        pidx_hbm.at[pl.ds(_XP_OFF + 2 * base, 2 * _CHUNK)], xpv, sem1)
    ca = pltpu.async_copy(ptab_hbm.at[pl.ds(0, 64)], av, sem2)
    cb = pltpu.async_copy(ptab_hbm.at[pl.ds(64, _L)], bv, sem3)
    cb.wait()
    ca.wait()

    beta = bv[...]
    dgrid = lax.iota(jnp.int32, _L).astype(jnp.float32)
    ev[...] = beta * jnp.exp(-beta * dgrid)
    cx.wait()
    cxp.wait()

    def step(j, carry):
        r2 = 2 * (j * _L + lax.iota(jnp.int32, _L))
        x0 = plsc.load_gather(xv, [r2])
        x1 = plsc.load_gather(xv, [r2 + 1])
        xp0 = plsc.load_gather(xpv, [r2])
        xp1 = plsc.load_gather(xpv, [r2 + 1])
        dt = jnp.abs(x0 - xp0)
        a_ = plsc.load_gather(av, [xp1 * 8 + x1])
        e_ = plsc.load_gather(ev, [dt])
        outv[pl.ds(j * _L, _L)] = a_ * e_
        return carry

    lax.fori_loop(0, _CHUNK // _L, step, 0)
    pltpu.sync_copy(outv, out_hbm.at[pl.ds(base, _CHUNK)])


@functools.partial(
    pl.kernel,
    out_type=jax.ShapeDtypeStruct((_B,), jnp.float32),
    mesh=plsc.VectorSubcoreMesh(core_axis_name="c", subcore_axis_name="s"),
    compiler_params=pltpu.CompilerParams(
        needs_layout_passes=False, use_tc_tiling_on_sc=False),
    scratch_types=[
        pltpu.VMEM((2 * _CHUNK,), jnp.int32),
        pltpu.VMEM((2 * _CHUNK,), jnp.int32),
        pltpu.VMEM((64,), jnp.float32),
        pltpu.VMEM((_L,), jnp.float32),
        pltpu.VMEM((_L,), jnp.float32),
        pltpu.VMEM((_CHUNK,), jnp.float32),
        pltpu.SemaphoreType.DMA,
        pltpu.SemaphoreType.DMA,
        pltpu.SemaphoreType.DMA,
        pltpu.SemaphoreType.DMA,
    ],
)
def _sc_kernel(pidx_hbm, ptab_hbm, out_hbm, *scratch):
    _sc_body(pidx_hbm, ptab_hbm, out_hbm, *scratch)


def kernel(x, xp, alpha, beta):
    pidx = jnp.concatenate([x.reshape(-1), xp.reshape(-1)])
    ptab = jnp.concatenate(
        [alpha.reshape(-1), jnp.broadcast_to(beta, (_L,))])
    return _sc_kernel(pidx, ptab)


# speedup vs baseline: 1.0589x; 1.0589x over previous
import functools

import jax
import jax.numpy as jnp
from jax import lax
from jax.experimental import pallas as pl
from jax.experimental.pallas import tpu as pltpu
from jax.experimental.pallas import tpu_sc as plsc

_B = 16384
_NW = 32
_CHUNK = _B // _NW
_L = 16
_XP_OFF = 2 * _B


def _sc_body(pidx_hbm, ptab_hbm, out_hbm, xv, xpv, av, bv, ev, outv,
             sem0, sem1, sem2, sem3):
    wid = lax.axis_index("s") * 2 + lax.axis_index("c")
    base = wid * _CHUNK
    cx = pltpu.async_copy(pidx_hbm.at[pl.ds(2 * base, 2 * _CHUNK)], xv, sem0)
    cxp = pltpu.async_copy(
        pidx_hbm.at[pl.ds(_XP_OFF + 2 * base, 2 * _CHUNK)], xpv, sem1)
    ca = pltpu.async_copy(ptab_hbm.at[pl.ds(0, 64)], av, sem2)
    cb = pltpu.async_copy(ptab_hbm.at[pl.ds(64, _L)], bv, sem3)
    cb.wait()
    ca.wait()

    beta = bv[...]
    dgrid = lax.iota(jnp.int32, _L).astype(jnp.float32)
    ev[...] = beta * jnp.exp(-beta * dgrid)
    cx.wait()
    cxp.wait()

    def step(j, carry):
        r2 = 2 * (j * _L + lax.iota(jnp.int32, _L))
        x0 = plsc.load_gather(xv, [r2])
        x1 = plsc.load_gather(xv, [r2 + 1])
        xp0 = plsc.load_gather(xpv, [r2])
        xp1 = plsc.load_gather(xpv, [r2 + 1])
        dt = jnp.abs(x0 - xp0)
        a_ = plsc.load_gather(av, [xp1 * 8 + x1])
        e_ = plsc.load_gather(ev, [dt])
        outv[pl.ds(j * _L, _L)] = a_ * e_
        return carry

    lax.fori_loop(0, _CHUNK // _L, step, 0)
    pltpu.sync_copy(outv, out_hbm.at[pl.ds(base, _CHUNK)])


@functools.partial(
    pl.kernel,
    out_type=jax.ShapeDtypeStruct((_B,), jnp.float32),
    mesh=plsc.VectorSubcoreMesh(core_axis_name="c", subcore_axis_name="s"),
    compiler_params=pltpu.CompilerParams(
        needs_layout_passes=False, use_tc_tiling_on_sc=False),
    scratch_types=[
        pltpu.VMEM((2 * _CHUNK,), jnp.int32),
        pltpu.VMEM((2 * _CHUNK,), jnp.int32),
        pltpu.VMEM((64,), jnp.float32),
        pltpu.VMEM((_L,), jnp.float32),
        pltpu.VMEM((_L,), jnp.float32),
        pltpu.VMEM((_CHUNK,), jnp.float32),
        pltpu.SemaphoreType.DMA,
        pltpu.SemaphoreType.DMA,
        pltpu.SemaphoreType.DMA,
        pltpu.SemaphoreType.DMA,
    ],
)
def _sc_kernel(pidx_hbm, ptab_hbm, out_hbm, *scratch):
    _sc_body(pidx_hbm, ptab_hbm, out_hbm, *scratch)


def kernel(x, xp, alpha, beta):
    pidx = jnp.concatenate([x, xp], axis=0).reshape(-1)
    ptab = jnp.concatenate(
        [alpha.reshape(-1), jnp.broadcast_to(beta, (_L,))])
    return _sc_kernel(pidx, ptab)


# speedup vs baseline: 1.0783x; 1.0183x over previous
import functools

import jax
import jax.numpy as jnp
from jax import lax
from jax.experimental import pallas as pl
from jax.experimental.pallas import tpu as pltpu
from jax.experimental.pallas import tpu_sc as plsc

_B = 16384
_NW = 32
_CHUNK = _B // _NW
_L = 16
_XP_OFF = 2 * _B
_TAB_OFF = 4 * _B


def _sc_body(pidx_hbm, out_hbm, xv, xpv, av, bv, ev, outv,
             sem0, sem1, sem2, sem3):
    wid = lax.axis_index("s") * 2 + lax.axis_index("c")
    base = wid * _CHUNK
    cx = pltpu.async_copy(pidx_hbm.at[pl.ds(2 * base, 2 * _CHUNK)], xv, sem0)
    cxp = pltpu.async_copy(
        pidx_hbm.at[pl.ds(_XP_OFF + 2 * base, 2 * _CHUNK)], xpv, sem1)
    ca = pltpu.async_copy(pidx_hbm.at[pl.ds(_TAB_OFF, 64)], av, sem2)
    cb = pltpu.async_copy(pidx_hbm.at[pl.ds(_TAB_OFF + 64, _L)], bv, sem3)
    cb.wait()
    ca.wait()

    beta = plsc.bitcast(bv[...], jnp.float32)
    dgrid = lax.iota(jnp.int32, _L).astype(jnp.float32)
    ev[...] = beta * jnp.exp(-beta * dgrid)
    cx.wait()
    cxp.wait()

    def step(j, carry):
        r2 = 2 * (j * _L + lax.iota(jnp.int32, _L))
        x0 = plsc.load_gather(xv, [r2])
        x1 = plsc.load_gather(xv, [r2 + 1])
        xp0 = plsc.load_gather(xpv, [r2])
        xp1 = plsc.load_gather(xpv, [r2 + 1])
        dt = jnp.abs(x0 - xp0)
        a_ = plsc.bitcast(plsc.load_gather(av, [xp1 * 8 + x1]),
                          jnp.float32)
        e_ = plsc.load_gather(ev, [dt])
        outv[pl.ds(j * _L, _L)] = a_ * e_
        return carry

    lax.fori_loop(0, _CHUNK // _L, step, 0)
    pltpu.sync_copy(outv, out_hbm.at[pl.ds(base, _CHUNK)])


@functools.partial(
    pl.kernel,
    out_type=jax.ShapeDtypeStruct((_B,), jnp.float32),
    mesh=plsc.VectorSubcoreMesh(core_axis_name="c", subcore_axis_name="s"),
    compiler_params=pltpu.CompilerParams(
        needs_layout_passes=False, use_tc_tiling_on_sc=False),
    scratch_types=[
        pltpu.VMEM((2 * _CHUNK,), jnp.int32),
        pltpu.VMEM((2 * _CHUNK,), jnp.int32),
        pltpu.VMEM((64,), jnp.int32),
        pltpu.VMEM((_L,), jnp.int32),
        pltpu.VMEM((_L,), jnp.float32),
        pltpu.VMEM((_CHUNK,), jnp.float32),
        pltpu.SemaphoreType.DMA,
        pltpu.SemaphoreType.DMA,
        pltpu.SemaphoreType.DMA,
        pltpu.SemaphoreType.DMA,
    ],
)
def _sc_kernel(pidx_hbm, out_hbm, *scratch):
    _sc_body(pidx_hbm, out_hbm, *scratch)


def kernel(x, xp, alpha, beta):
    tab = jnp.concatenate(
        [alpha.reshape(-1), jnp.broadcast_to(beta, (_L,))])
    pidx = jnp.concatenate([
        jnp.concatenate([x, xp], axis=0).reshape(-1),
        lax.bitcast_convert_type(tab, jnp.int32),
    ])
    return _sc_kernel(pidx)
